# Initial kernel scaffold; baseline (speedup 1.0000x reference)
import functools

import jax
import jax.numpy as jnp
import numpy as np
from jax import lax
from jax.experimental import pallas as pl
from jax.experimental.pallas import tpu as pltpu
from jax.experimental.pallas import tpu_sc as plsc

N = 9984
F = 128
G = 78
E = 319488
HC = 128
HID = 256
NCLS = 2
IN_DIM = F * (F - 1) // 2
BNC = float(1.0 / np.sqrt(1.0 + 1e-5))

NW = 32
EPW = E // NW
EV = EPW // 16
NV = N // 16

KBLK = 2048
KB = (F * F) // KBLK


@functools.partial(
    pl.kernel,
    out_type=jax.ShapeDtypeStruct((NW, N), jnp.float32),
    mesh=plsc.VectorSubcoreMesh(core_axis_name="c", subcore_axis_name="s"),
    scratch_types=[
        pltpu.VMEM((EPW,), jnp.int32),
        pltpu.VMEM((N,), jnp.float32),
    ],
)
def _deg_partials(dst_hbm, out_hbm, idx_v, hist_v):
    c = lax.axis_index("c")
    s = lax.axis_index("s")
    w = s * 2 + c

    pltpu.sync_copy(dst_hbm.at[pl.ds(w * EPW, EPW)], idx_v)

    zeros = jnp.zeros((16,), jnp.float32)

    def zero_body(i, carry):
        hist_v[pl.ds(i * 16, 16)] = zeros
        return carry

    lax.fori_loop(0, NV, zero_body, 0)

    def hist_body(i, carry):
        idx = idx_v[pl.ds(i * 16, 16)]
        cnt, last = plsc.scan_count(idx)
        plsc.addupdate_scatter(hist_v, [idx], cnt.astype(jnp.float32),
                               mask=last)
        return carry

    lax.fori_loop(0, EV, hist_body, 0)

    pltpu.sync_copy(hist_v, out_hbm.at[w])


def _conv_pool_body(x_ref, dp_ref, w0t_ref, b0_ref, w1t_ref, b1_ref, h_ref):
    d = jnp.sum(dp_ref[...], axis=0, keepdims=True) + 1.0
    a = jnp.dot(x_ref[...], w0t_ref[...],
                preferred_element_type=jnp.float32) + b0_ref[...]
    dd = jnp.concatenate([d, d * d], axis=0)
    t = jnp.dot(dd, a, preferred_element_type=jnp.float32)
    h1 = t[0:1, :] * (1.0 / F)
    h2 = (jnp.dot(t[1:2, :], w1t_ref[...],
                  preferred_element_type=jnp.float32)
          + jnp.sum(d) * b1_ref[...]) * (1.0 / F)
    h_ref[...] = jnp.concatenate([h1, h2], axis=1)


def _conv_pool(x, deg_parts, w0t, b0, w1t, b1):
    return pl.pallas_call(
        _conv_pool_body,
        grid=(G,),
        in_specs=[
            pl.BlockSpec((F, F), lambda g: (g, 0)),
            pl.BlockSpec((NW, F), lambda g: (0, g)),
            pl.BlockSpec((F, HC), lambda g: (0, 0)),
            pl.BlockSpec((1, HC), lambda g: (0, 0)),
            pl.BlockSpec((HC, HC), lambda g: (0, 0)),
            pl.BlockSpec((1, HC), lambda g: (0, 0)),
        ],
        out_specs=pl.BlockSpec((1, 2 * HC), lambda g: (g, 0)),
        out_shape=jax.ShapeDtypeStruct((G, 2 * HC), jnp.float32),
    )(x, deg_parts, w0t, b0, w1t, b1)


def _mlp_body(xf_ref, wfull_ref, h_ref, wb_ref, beta1_ref, w2t_ref,
              beta2_ref, w3t_ref, beta3_ref, w4t_ref, b4_ref, out_ref,
              acc_ref):
    k = pl.program_id(0)
    part = jnp.dot(xf_ref[...], wfull_ref[...],
                   preferred_element_type=jnp.float32)

    @pl.when(k == 0)
    def _():
        acc_ref[...] = part

    @pl.when(k > 0)
    def _():
        acc_ref[...] += part

    @pl.when(k == KB - 1)
    def _():
        v = (acc_ref[...]
             + jnp.dot(h_ref[...], wb_ref[...],
                       preferred_element_type=jnp.float32)
             + beta1_ref[...])
        z1 = jnp.maximum(v, 0.0)
        z2 = jnp.maximum(
            jnp.dot(z1, w2t_ref[...], preferred_element_type=jnp.float32)
            + beta2_ref[...], 0.0)
        z3 = jnp.maximum(
            jnp.dot(z2, w3t_ref[...], preferred_element_type=jnp.float32)
            + beta3_ref[...], 0.0)
        out_ref[...] = (jnp.dot(z3, w4t_ref[...],
                                preferred_element_type=jnp.float32)
                        + b4_ref[...])


def _mlp(xf, wfull, hpre, wb, beta1, w2t, beta2, w3t, beta3, w4t, b4):
    return pl.pallas_call(
        _mlp_body,
        grid=(KB,),
        in_specs=[
            pl.BlockSpec((G, KBLK), lambda k: (0, k)),
            pl.BlockSpec((KBLK, HID), lambda k: (k, 0)),
            pl.BlockSpec((G, HID), lambda k: (0, 0)),
            pl.BlockSpec((HID, HID), lambda k: (0, 0)),
            pl.BlockSpec((1, HID), lambda k: (0, 0)),
            pl.BlockSpec((HID, HID // 2), lambda k: (0, 0)),
            pl.BlockSpec((1, HID // 2), lambda k: (0, 0)),
            pl.BlockSpec((HID // 2, HID // 2), lambda k: (0, 0)),
            pl.BlockSpec((1, HID // 2), lambda k: (0, 0)),
            pl.BlockSpec((HID // 2, HID // 2), lambda k: (0, 0)),
            pl.BlockSpec((1, HID // 2), lambda k: (0, 0)),
        ],
        out_specs=pl.BlockSpec((G, HID // 2), lambda k: (0, 0)),
        out_shape=jax.ShapeDtypeStruct((G, HID // 2), jnp.float32),
        scratch_shapes=[pltpu.VMEM((G, HID), jnp.float32)],
    )(xf, wfull, hpre, wb, beta1, w2t, beta2, w3t, beta3, w4t, b4)


_IU0, _IU1 = np.triu_indices(F, k=1)


def kernel(x, edge_index, batch, params):
    dst = edge_index[1]
    deg_parts = _deg_partials(dst)

    convs = params['convs']
    w0t = convs[0]['Wdst'].T
    b0 = convs[0]['bdst'].reshape(1, HC)
    w1t = convs[1]['Wdst'].T
    b1 = convs[1]['bdst'].reshape(1, HC)

    hpre = _conv_pool(x, deg_parts, w0t, b0, w1t, b1)

    c = BNC
    w1 = params['W1']
    w1a = w1[:, :IN_DIM]
    w1b = w1[:, IN_DIM:]
    g1c = (c * params['g1'])[None, :]
    wa = (w1a * (c * params['bn_g'])[None, :]).T
    wfull = (jnp.zeros((F, F, HID), jnp.float32)
             .at[_IU0, _IU1].set(wa)
             .reshape(F * F, HID)) * g1c
    wb = (w1b * (c * params['bnh_g'])[None, :]).T * g1c
    bias1 = (params['b1'] + w1a @ params['bn_b'] + w1b @ params['bnh_b'])
    beta1 = (g1c[0] * bias1 + params['be1']).reshape(1, HID)
    w2t = params['W2'].T * (c * params['g2'])[None, :]
    beta2 = (c * params['g2'] * params['b2'] + params['be2']).reshape(1, HID // 2)
    w3t = params['W3'].T * (c * params['g3'])[None, :]
    beta3 = (c * params['g3'] * params['b3'] + params['be3']).reshape(1, HID // 2)
    w4t = jnp.zeros((HID // 2, HID // 2), jnp.float32).at[:, :NCLS].set(
        params['W4'].T)
    b4 = jnp.zeros((1, HID // 2), jnp.float32).at[0, :NCLS].set(params['b4'])

    out_full = _mlp(x.reshape(G, F * F), wfull, hpre, wb, beta1,
                    w2t, beta2, w3t, beta3, w4t, b4)
    return out_full[:, :NCLS]


# speedup vs baseline: 28.0926x; 28.0926x over previous
import functools

import jax
import jax.numpy as jnp
import numpy as np
from jax import lax
from jax.experimental import pallas as pl
from jax.experimental.pallas import tpu as pltpu
from jax.experimental.pallas import tpu_sc as plsc

N = 9984
F = 128
G = 78
E = 319488
HC = 128
HID = 256
NCLS = 2
IN_DIM = F * (F - 1) // 2
BNC = float(1.0 / np.sqrt(1.0 + 1e-5))

NW = 32
EPW = E // NW
EV = EPW // 16
NV = N // 16

KBLK = 2048
KB = (F * F) // KBLK


def _deg_partials_body(dst_hbm, out_hbm, idx_v, hist_v):
    c = lax.axis_index("c")
    s = lax.axis_index("s")
    w = s * 2 + c

    pltpu.sync_copy(dst_hbm.at[pl.ds(w * EPW, EPW)], idx_v)

    zeros = jnp.zeros((16,), jnp.float32)

    def zero_body(i, carry):
        hist_v[pl.ds(i * 16, 16)] = zeros
        return carry

    lax.fori_loop(0, NV, zero_body, 0)

    def hist_body(i, carry):
        idx = idx_v[pl.ds(i * 16, 16)]
        cnt, last = plsc.scan_count(idx)
        plsc.addupdate_scatter(hist_v, [idx], cnt.astype(jnp.float32),
                               mask=last)
        return carry

    lax.fori_loop(0, EV, hist_body, 0)

    pltpu.sync_copy(hist_v, out_hbm.at[w])


@functools.cache
def _deg_partials():
    return pl.kernel(
        _deg_partials_body,
        out_type=jax.ShapeDtypeStruct((NW, N), jnp.float32),
        mesh=plsc.VectorSubcoreMesh(core_axis_name="c", subcore_axis_name="s"),
        scratch_types=[
            pltpu.VMEM((EPW,), jnp.int32),
            pltpu.VMEM((N,), jnp.float32),
        ],
        compiler_params=pltpu.CompilerParams(needs_layout_passes=False),
    )


def _conv_pool_body(x_ref, dp_ref, w0t_ref, b0_ref, w1t_ref, b1_ref, h_ref):
    d = jnp.sum(dp_ref[...], axis=0, keepdims=True) + 1.0
    a = jnp.dot(x_ref[...], w0t_ref[...],
                preferred_element_type=jnp.float32) + b0_ref[...]
    dd = jnp.concatenate([d, d * d], axis=0)
    t = jnp.dot(dd, a, preferred_element_type=jnp.float32)
    h1 = t[0:1, :] * (1.0 / F)
    h2 = (jnp.dot(t[1:2, :], w1t_ref[...],
                  preferred_element_type=jnp.float32)
          + jnp.sum(d) * b1_ref[...]) * (1.0 / F)
    h_ref[...] = jnp.concatenate([h1, h2], axis=1).reshape(1, 1, 2 * HC)


def _conv_pool(x, deg_parts, w0t, b0, w1t, b1):
    return pl.pallas_call(
        _conv_pool_body,
        grid=(G,),
        in_specs=[
            pl.BlockSpec((F, F), lambda g: (g, 0)),
            pl.BlockSpec((NW, F), lambda g: (0, g)),
            pl.BlockSpec((F, HC), lambda g: (0, 0)),
            pl.BlockSpec((1, HC), lambda g: (0, 0)),
            pl.BlockSpec((HC, HC), lambda g: (0, 0)),
            pl.BlockSpec((1, HC), lambda g: (0, 0)),
        ],
        out_specs=pl.BlockSpec((1, 1, 2 * HC), lambda g: (g, 0, 0)),
        out_shape=jax.ShapeDtypeStruct((G, 1, 2 * HC), jnp.float32),
    )(x, deg_parts, w0t, b0, w1t, b1).reshape(G, 2 * HC)


def _mlp_body(xf_ref, wfull_ref, h_ref, wb_ref, beta1_ref, w2t_ref,
              beta2_ref, w3t_ref, beta3_ref, w4t_ref, b4_ref, out_ref,
              acc_ref):
    k = pl.program_id(0)
    part = jnp.dot(xf_ref[...], wfull_ref[...],
                   preferred_element_type=jnp.float32)

    @pl.when(k == 0)
    def _():
        acc_ref[...] = part

    @pl.when(k > 0)
    def _():
        acc_ref[...] += part

    @pl.when(k == KB - 1)
    def _():
        v = (acc_ref[...]
             + jnp.dot(h_ref[...], wb_ref[...],
                       preferred_element_type=jnp.float32)
             + beta1_ref[...])
        z1 = jnp.maximum(v, 0.0)
        z2 = jnp.maximum(
            jnp.dot(z1, w2t_ref[...], preferred_element_type=jnp.float32)
            + beta2_ref[...], 0.0)
        z3 = jnp.maximum(
            jnp.dot(z2, w3t_ref[...], preferred_element_type=jnp.float32)
            + beta3_ref[...], 0.0)
        out_ref[...] = (jnp.dot(z3, w4t_ref[...],
                                preferred_element_type=jnp.float32)
                        + b4_ref[...])


def _mlp(xf, wfull, hpre, wb, beta1, w2t, beta2, w3t, beta3, w4t, b4):
    return pl.pallas_call(
        _mlp_body,
        grid=(KB,),
        in_specs=[
            pl.BlockSpec((G, KBLK), lambda k: (0, k)),
            pl.BlockSpec((KBLK, HID), lambda k: (k, 0)),
            pl.BlockSpec((G, HID), lambda k: (0, 0)),
            pl.BlockSpec((HID, HID), lambda k: (0, 0)),
            pl.BlockSpec((1, HID), lambda k: (0, 0)),
            pl.BlockSpec((HID, HID // 2), lambda k: (0, 0)),
            pl.BlockSpec((1, HID // 2), lambda k: (0, 0)),
            pl.BlockSpec((HID // 2, HID // 2), lambda k: (0, 0)),
            pl.BlockSpec((1, HID // 2), lambda k: (0, 0)),
            pl.BlockSpec((HID // 2, HID // 2), lambda k: (0, 0)),
            pl.BlockSpec((1, HID // 2), lambda k: (0, 0)),
        ],
        out_specs=pl.BlockSpec((G, HID // 2), lambda k: (0, 0)),
        out_shape=jax.ShapeDtypeStruct((G, HID // 2), jnp.float32),
        scratch_shapes=[pltpu.VMEM((G, HID), jnp.float32)],
    )(xf, wfull, hpre, wb, beta1, w2t, beta2, w3t, beta3, w4t, b4)


_IU0, _IU1 = np.triu_indices(F, k=1)


def kernel(x, edge_index, batch, params):
    dst = edge_index[1]
    deg_parts = _deg_partials()(dst)

    convs = params['convs']
    w0t = convs[0]['Wdst'].T
    b0 = convs[0]['bdst'].reshape(1, HC)
    w1t = convs[1]['Wdst'].T
    b1 = convs[1]['bdst'].reshape(1, HC)

    hpre = _conv_pool(x, deg_parts, w0t, b0, w1t, b1)

    c = BNC
    w1 = params['W1']
    w1a = w1[:, :IN_DIM]
    w1b = w1[:, IN_DIM:]
    g1c = (c * params['g1'])[None, :]
    wa = (w1a * (c * params['bn_g'])[None, :]).T
    wfull = (jnp.zeros((F, F, HID), jnp.float32)
             .at[_IU0, _IU1].set(wa)
             .reshape(F * F, HID)) * g1c
    wb = (w1b * (c * params['bnh_g'])[None, :]).T * g1c
    bias1 = (params['b1'] + w1a @ params['bn_b'] + w1b @ params['bnh_b'])
    beta1 = (g1c[0] * bias1 + params['be1']).reshape(1, HID)
    w2t = params['W2'].T * (c * params['g2'])[None, :]
    beta2 = (c * params['g2'] * params['b2'] + params['be2']).reshape(1, HID // 2)
    w3t = params['W3'].T * (c * params['g3'])[None, :]
    beta3 = (c * params['g3'] * params['b3'] + params['be3']).reshape(1, HID // 2)
    w4t = jnp.zeros((HID // 2, HID // 2), jnp.float32).at[:, :NCLS].set(
        params['W4'].T)
    b4 = jnp.zeros((1, HID // 2), jnp.float32).at[0, :NCLS].set(params['b4'])

    out_full = _mlp(x.reshape(G, F * F), wfull, hpre, wb, beta1,
                    w2t, beta2, w3t, beta3, w4t, b4)
    return out_full[:, :NCLS]


# speedup vs baseline: 82.3612x; 2.9318x over previous
import functools

import jax
import jax.numpy as jnp
import numpy as np
from jax import lax
from jax.experimental import pallas as pl
from jax.experimental.pallas import tpu as pltpu
from jax.experimental.pallas import tpu_sc as plsc

N = 9984
F = 128
G = 78
E = 319488
HC = 128
HID = 256
NCLS = 2
IN_DIM = F * (F - 1) // 2
BNC = float(1.0 / np.sqrt(1.0 + 1e-5))

NW = 32
EPW = E // NW
EV = EPW // 16
NV = N // 16


def _deg_partials_body(dst_hbm, out_hbm, idx_v, hist_v):
    c = lax.axis_index("c")
    s = lax.axis_index("s")
    w = s * 2 + c

    pltpu.sync_copy(dst_hbm.at[pl.ds(w * EPW, EPW)], idx_v)

    zeros = jnp.zeros((16,), jnp.float32)

    def zero_body(i, carry):
        hist_v[pl.ds(i * 16, 16)] = zeros
        return carry

    lax.fori_loop(0, NV, zero_body, 0)

    def hist_body(i, carry):
        idx = idx_v[pl.ds(i * 16, 16)]
        cnt, last = plsc.scan_count(idx)
        plsc.addupdate_scatter(hist_v, [idx], cnt.astype(jnp.float32),
                               mask=last)
        return carry

    lax.fori_loop(0, EV, hist_body, 0)

    pltpu.sync_copy(hist_v, out_hbm.at[w])


@functools.cache
def _deg_partials():
    return pl.kernel(
        _deg_partials_body,
        out_type=jax.ShapeDtypeStruct((NW, N), jnp.float32),
        mesh=plsc.VectorSubcoreMesh(core_axis_name="c", subcore_axis_name="s"),
        scratch_types=[
            pltpu.VMEM((EPW,), jnp.int32),
            pltpu.VMEM((N,), jnp.float32),
        ],
        compiler_params=pltpu.CompilerParams(needs_layout_passes=False),
    )


def _conv_pool_body(x_ref, dp_ref, w0t_ref, b0_ref, w1t_ref, b1_ref, h_ref):
    d = jnp.sum(dp_ref[...], axis=0, keepdims=True) + 1.0
    a = jnp.dot(x_ref[...], w0t_ref[...],
                preferred_element_type=jnp.float32) + b0_ref[...]
    rowid = lax.broadcasted_iota(jnp.int32, (2 * G, N), 0)
    colg = lax.broadcasted_iota(jnp.int32, (2 * G, N), 1) // F
    b = jnp.where(rowid < G, rowid, rowid - G)
    base = jnp.where(rowid < G, d, d * d)
    md = jnp.where(colg == b, base, 0.0)
    t = jnp.dot(md, a, preferred_element_type=jnp.float32)
    h1 = t[:G] * (1.0 / F)
    sumd = jnp.sum(md[:G], axis=1, keepdims=True)
    h2 = (jnp.dot(t[G:], w1t_ref[...],
                  preferred_element_type=jnp.float32)
          + sumd * b1_ref[...]) * (1.0 / F)
    h_ref[...] = jnp.concatenate([h1, h2], axis=1)


def _conv_pool(x, deg_parts, w0t, b0, w1t, b1):
    return pl.pallas_call(
        _conv_pool_body,
        out_shape=jax.ShapeDtypeStruct((G, 2 * HC), jnp.float32),
    )(x, deg_parts, w0t, b0, w1t, b1)


_OFF = [i * (F - 1) - i * (i - 1) // 2 for i in range(F)]
WPAD = 8192


def _mlp_body(xf_ref, w1at_ref, h_ref, wb_ref, beta1_ref, w2t_ref,
              beta2_ref, w3t_ref, beta3_ref, w4t_ref, b4_ref, out_ref,
              wf_ref):
    rows = lax.broadcasted_iota(jnp.int32, (F, HID), 0)
    for i in range(F):
        s = 8 + _OFF[i] - (i + 1)
        chunk = w1at_ref[pl.ds(s, F), :]
        wf_ref[pl.ds(i * F, F), :] = jnp.where(rows > i, chunk, 0.0)

    v = (jnp.dot(xf_ref[...], wf_ref[...],
                 preferred_element_type=jnp.float32)
         + jnp.dot(h_ref[...], wb_ref[...],
                   preferred_element_type=jnp.float32)
         + beta1_ref[...])
    z1 = jnp.maximum(v, 0.0)
    z2 = jnp.maximum(
        jnp.dot(z1, w2t_ref[...], preferred_element_type=jnp.float32)
        + beta2_ref[...], 0.0)
    z3 = jnp.maximum(
        jnp.dot(z2, w3t_ref[...], preferred_element_type=jnp.float32)
        + beta3_ref[...], 0.0)
    out_ref[...] = (jnp.dot(z3, w4t_ref[...],
                            preferred_element_type=jnp.float32)
                    + b4_ref[...])


def _mlp(xf, w1atp, hpre, wb, beta1, w2t, beta2, w3t, beta3, w4t, b4):
    return pl.pallas_call(
        _mlp_body,
        out_shape=jax.ShapeDtypeStruct((G, HID // 2), jnp.float32),
        scratch_shapes=[pltpu.VMEM((F * F, HID), jnp.float32)],
    )(xf, w1atp, hpre, wb, beta1, w2t, beta2, w3t, beta3, w4t, b4)


def kernel(x, edge_index, batch, params):
    dst = edge_index[1]
    deg_parts = _deg_partials()(dst)

    convs = params['convs']
    w0t = convs[0]['Wdst'].T
    b0 = convs[0]['bdst'].reshape(1, HC)
    w1t = convs[1]['Wdst'].T
    b1 = convs[1]['bdst'].reshape(1, HC)

    hpre = _conv_pool(x, deg_parts, w0t, b0, w1t, b1)

    c = BNC
    w1 = params['W1']
    w1a = w1[:, :IN_DIM]
    w1b = w1[:, IN_DIM:]
    g1c = (c * params['g1'])[None, :]
    wa = (w1a * (c * params['bn_g'])[None, :]).T * g1c
    w1atp = jnp.zeros((WPAD, HID), jnp.float32).at[8:8 + IN_DIM].set(wa)
    wb = (w1b * (c * params['bnh_g'])[None, :]).T * g1c
    bias1 = (params['b1'] + w1a @ params['bn_b'] + w1b @ params['bnh_b'])
    beta1 = (g1c[0] * bias1 + params['be1']).reshape(1, HID)
    w2t = params['W2'].T * (c * params['g2'])[None, :]
    beta2 = (c * params['g2'] * params['b2'] + params['be2']).reshape(1, HID // 2)
    w3t = params['W3'].T * (c * params['g3'])[None, :]
    beta3 = (c * params['g3'] * params['b3'] + params['be3']).reshape(1, HID // 2)
    w4t = jnp.zeros((HID // 2, HID // 2), jnp.float32).at[:, :NCLS].set(
        params['W4'].T)
    b4 = jnp.zeros((1, HID // 2), jnp.float32).at[0, :NCLS].set(params['b4'])

    out_full = _mlp(x.reshape(G, F * F), w1atp, hpre, wb, beta1,
                    w2t, beta2, w3t, beta3, w4t, b4)
    return out_full[:, :NCLS]


# speedup vs baseline: 86.4972x; 1.0502x over previous
import functools

import jax
import jax.numpy as jnp
import numpy as np
from jax import lax
from jax.experimental import pallas as pl
from jax.experimental.pallas import tpu as pltpu
from jax.experimental.pallas import tpu_sc as plsc

N = 9984
F = 128
G = 78
E = 319488
HC = 128
HID = 256
NCLS = 2
IN_DIM = F * (F - 1) // 2
BNC = float(1.0 / np.sqrt(1.0 + 1e-5))

NW = 32
EPW = E // NW
EV = EPW // 16
NV = N // 16


def _deg_partials_body(edge_hbm, out_hbm, idx_v, hist_v):
    c = lax.axis_index("c")
    s = lax.axis_index("s")
    w = s * 2 + c

    pltpu.sync_copy(edge_hbm.at[1, pl.ds(w * EPW, EPW)], idx_v)

    zeros = jnp.zeros((16,), jnp.float32)

    def zero_body(i, carry):
        for u in range(8):
            hist_v[pl.ds(i * 128 + u * 16, 16)] = zeros
        return carry

    lax.fori_loop(0, NV // 8, zero_body, 0)

    def hist_body(i, carry):
        for u in range(4):
            idx = idx_v[pl.ds(i * 64 + u * 16, 16)]
            cnt, last = plsc.scan_count(idx)
            plsc.addupdate_scatter(hist_v, [idx], cnt.astype(jnp.float32),
                                   mask=last)
        return carry

    lax.fori_loop(0, EV // 4, hist_body, 0)

    pltpu.sync_copy(hist_v, out_hbm.at[w])


@functools.cache
def _deg_partials():
    return pl.kernel(
        _deg_partials_body,
        out_type=jax.ShapeDtypeStruct((NW, N), jnp.float32),
        mesh=plsc.VectorSubcoreMesh(core_axis_name="c", subcore_axis_name="s"),
        scratch_types=[
            pltpu.VMEM((EPW,), jnp.int32),
            pltpu.VMEM((N,), jnp.float32),
        ],
        compiler_params=pltpu.CompilerParams(needs_layout_passes=False),
    )


def _conv_pool_body(x_ref, dp_ref, w0t_ref, b0_ref, w1t_ref, b1_ref, h_ref):
    d = jnp.sum(dp_ref[...], axis=0, keepdims=True) + 1.0
    a = jnp.dot(x_ref[...], w0t_ref[...],
                preferred_element_type=jnp.float32) + b0_ref[...]
    rowid = lax.broadcasted_iota(jnp.int32, (2 * G, N), 0)
    colg = lax.broadcasted_iota(jnp.int32, (2 * G, N), 1) // F
    b = jnp.where(rowid < G, rowid, rowid - G)
    base = jnp.where(rowid < G, d, d * d)
    md = jnp.where(colg == b, base, 0.0)
    t = jnp.dot(md, a, preferred_element_type=jnp.float32)
    h1 = t[:G] * (1.0 / F)
    sumd = jnp.sum(md[:G], axis=1, keepdims=True)
    h2 = (jnp.dot(t[G:], w1t_ref[...],
                  preferred_element_type=jnp.float32)
          + sumd * b1_ref[...]) * (1.0 / F)
    h_ref[...] = jnp.concatenate([h1, h2], axis=1)


def _conv_pool(x, deg_parts, w0t, b0, w1t, b1):
    return pl.pallas_call(
        _conv_pool_body,
        out_shape=jax.ShapeDtypeStruct((G, 2 * HC), jnp.float32),
    )(x, deg_parts, w0t, b0, w1t, b1)


_OFF = [i * (F - 1) - i * (i - 1) // 2 for i in range(F)]
WTPAD = 8192


def _mlp_body(xf_ref, w1_ref, bn_ref, h_ref, bnh_ref, g1_ref, beta1_ref,
              w2t_ref, beta2_ref, w3t_ref, beta3_ref, w4t_ref, b4_ref,
              out_ref, w1at_ref, wf_ref):
    w1at_ref[0:8, :] = jnp.zeros((8, HID), jnp.float32)
    w1at_ref[8 + IN_DIM:, :] = jnp.zeros((WTPAD - 8 - IN_DIM, HID),
                                         jnp.float32)
    w1at_ref[8:8 + IN_DIM, :] = jnp.transpose(
        w1_ref[:, :IN_DIM] * bn_ref[...])

    rows = lax.broadcasted_iota(jnp.int32, (F, HID), 0)
    for i in range(F):
        s = 8 + _OFF[i] - (i + 1)
        chunk = w1at_ref[pl.ds(s, F), :]
        wf_ref[pl.ds(i * F, F), :] = jnp.where(rows > i, chunk, 0.0)

    hq = h_ref[...] * bnh_ref[...]
    nd = (((1,), (1,)), ((), ()))
    v = (jnp.dot(xf_ref[...], wf_ref[...],
                 preferred_element_type=jnp.float32)
         + lax.dot_general(hq, w1_ref[:, IN_DIM:], nd,
                           preferred_element_type=jnp.float32))
    v = v * g1_ref[...] + beta1_ref[...]
    z1 = jnp.maximum(v, 0.0)
    z2 = jnp.maximum(
        jnp.dot(z1, w2t_ref[...], preferred_element_type=jnp.float32)
        + beta2_ref[...], 0.0)
    z3 = jnp.maximum(
        jnp.dot(z2, w3t_ref[...], preferred_element_type=jnp.float32)
        + beta3_ref[...], 0.0)
    out_ref[...] = (jnp.dot(z3, w4t_ref[...],
                            preferred_element_type=jnp.float32)
                    + b4_ref[...])


def _mlp(xf, w1, bn, hpre, bnh, g1r, beta1, w2t, beta2, w3t, beta3, w4t, b4):
    return pl.pallas_call(
        _mlp_body,
        out_shape=jax.ShapeDtypeStruct((G, HID // 2), jnp.float32),
        scratch_shapes=[pltpu.VMEM((WTPAD, HID), jnp.float32),
                        pltpu.VMEM((F * F, HID), jnp.float32)],
    )(xf, w1, bn, hpre, bnh, g1r, beta1, w2t, beta2, w3t, beta3, w4t, b4)


def kernel(x, edge_index, batch, params):
    deg_parts = _deg_partials()(edge_index)

    convs = params['convs']
    w0t = convs[0]['Wdst'].T
    b0 = convs[0]['bdst'].reshape(1, HC)
    w1t = convs[1]['Wdst'].T
    b1 = convs[1]['bdst'].reshape(1, HC)

    hpre = _conv_pool(x, deg_parts, w0t, b0, w1t, b1)

    c = BNC
    w1 = params['W1']
    w1a = w1[:, :IN_DIM]
    w1b = w1[:, IN_DIM:]
    bn = (c * params['bn_g']).reshape(1, IN_DIM)
    bnh = (c * params['bnh_g']).reshape(1, HID)
    g1r = (c * params['g1']).reshape(1, HID)
    bias1 = (params['b1'] + w1a @ params['bn_b'] + w1b @ params['bnh_b'])
    beta1 = (g1r[0] * bias1 + params['be1']).reshape(1, HID)
    w2t = params['W2'].T * (c * params['g2'])[None, :]
    beta2 = (c * params['g2'] * params['b2'] + params['be2']).reshape(1, HID // 2)
    w3t = params['W3'].T * (c * params['g3'])[None, :]
    beta3 = (c * params['g3'] * params['b3'] + params['be3']).reshape(1, HID // 2)
    w4t = jnp.zeros((HID // 2, HID // 2), jnp.float32).at[:, :NCLS].set(
        params['W4'].T)
    b4 = jnp.zeros((1, HID // 2), jnp.float32).at[0, :NCLS].set(params['b4'])

    out_full = _mlp(x.reshape(G, F * F), w1, bn, hpre, bnh, g1r, beta1,
                    w2t, beta2, w3t, beta3, w4t, b4)
    return out_full[:, :NCLS]
